# baseline (speedup 1.0000x reference)
import jax
import jax.numpy as jnp
from jax import lax
from jax.experimental import pallas as pl

NB, NA, NH, NW, NCLS = 16, 3, 64, 64, 80
STRIDE = 8.0
ANCH_W = (10.0, 16.0, 33.0)
ANCH_H = (13.0, 30.0, 23.0)
SUB = 32
LANES = 128


def _decode_body(rb_ref, rc_ref, rcls_ref, obox_ref, oidx_ref, oscore_ref):
    a = pl.program_id(1)

    t = rb_ref[0, 0]
    lane = lax.broadcasted_iota(jnp.int32, (NH, NW * 4), 1)
    c = lane % 4
    wf = (lane // 4).astype(jnp.float32)
    row = lax.broadcasted_iota(jnp.int32, (NH, NW * 4), 0).astype(jnp.float32)
    sig = jax.nn.sigmoid(t)
    ex = jnp.exp(t)
    aw = jnp.where(a == 0, ANCH_W[0], jnp.where(a == 1, ANCH_W[1], ANCH_W[2]))
    ah = jnp.where(a == 0, ANCH_H[0], jnp.where(a == 1, ANCH_H[1], ANCH_H[2]))
    xy = (sig + jnp.where(c == 0, wf, row)) * STRIDE
    wh = ex * jnp.where(c == 2, aw, ah)
    obox_ref[0, 0] = jnp.where(c < 2, xy, wh)

    tc = rcls_ref[0, 0]
    m = jnp.max(tc, axis=2, keepdims=True)
    li = lax.broadcasted_iota(jnp.int32, (SUB, LANES, NCLS), 2)
    idx = jnp.min(jnp.where(tc == m, li, NCLS), axis=2)
    oidx_ref[0, 0] = idx
    conf = rc_ref[0, 0]
    oscore_ref[0, 0] = jax.nn.sigmoid(m[:, :, 0]) * jax.nn.sigmoid(conf)


def kernel(raw_bbox, raw_conf, raw_class, img_size):
    del img_size
    rb = raw_bbox.reshape(NB, NA, NH, NW * 4)
    rc = raw_conf.reshape(NB, NA, SUB, LANES)
    rcls = raw_class.reshape(NB, NA, SUB, LANES, NCLS)

    obox, oidx, oscore = pl.pallas_call(
        _decode_body,
        grid=(NB, NA),
        in_specs=[
            pl.BlockSpec((1, 1, NH, NW * 4), lambda b, a: (b, a, 0, 0)),
            pl.BlockSpec((1, 1, SUB, LANES), lambda b, a: (b, a, 0, 0)),
            pl.BlockSpec((1, 1, SUB, LANES, NCLS), lambda b, a: (b, a, 0, 0, 0)),
        ],
        out_specs=[
            pl.BlockSpec((1, 1, NH, NW * 4), lambda b, a: (b, a, 0, 0)),
            pl.BlockSpec((1, 1, SUB, LANES), lambda b, a: (b, a, 0, 0)),
            pl.BlockSpec((1, 1, SUB, LANES), lambda b, a: (b, a, 0, 0)),
        ],
        out_shape=[
            jax.ShapeDtypeStruct((NB, NA, NH, NW * 4), jnp.float32),
            jax.ShapeDtypeStruct((NB, NA, SUB, LANES), jnp.int32),
            jax.ShapeDtypeStruct((NB, NA, SUB, LANES), jnp.float32),
        ],
    )(rb, rc, rcls)

    preds_bbox = obox.reshape(NB, NA * NH * NW, 4)
    preds_class_idx = oidx.reshape(NB, NA * NH * NW)
    preds_score = oscore.reshape(NB, NA * NH * NW)
    return (preds_bbox, preds_class_idx, preds_score)
